# Initial kernel scaffold; baseline (speedup 1.0000x reference)
import jax
import jax.numpy as jnp
from jax.experimental import pallas as pl


def kernel(inputs, embedding_weight):
    raise NotImplementedError("write your pallas kernel here")


# speedup vs baseline: 10.1441x; 10.1441x over previous
import functools

import jax
import jax.numpy as jnp
from jax import lax
from jax.experimental import pallas as pl
from jax.experimental.pallas import tpu as pltpu
from jax.experimental.pallas import tpu_sc as plsc

_K = 8192
_D = 32
_N = 32768
_TB = 256
_GRID = _N // _TB
_COMMIT = 0.25

_SC_CORES = 2
_SC_SUBCORES = 16
_SC_WORKERS = _SC_CORES * _SC_SUBCORES
_BPW = _N // _SC_WORKERS
_CHUNK = 128


def _argmin_body(x_ref, x2_ref, emb_ref, e2_ref,
                 idx_ref, loss_ref, perp_ref, counts_ref, acc_ref):
    i = pl.program_id(0)

    @pl.when(i == 0)
    def _init():
        counts_ref[...] = jnp.zeros_like(counts_ref)
        acc_ref[0] = 0.0

    x = x_ref[...].astype(jnp.bfloat16).astype(jnp.float32)
    emb = emb_ref[...]
    mm = lax.dot_general(x, emb, (((1,), (1,)), ((), ())),
                         preferred_element_type=jnp.float32)
    d = (x2_ref[...] + e2_ref[...]) - 2.0 * mm
    dmin = jnp.min(d, axis=1)
    col = lax.broadcasted_iota(jnp.int32, (_TB, _K), 1)
    am = jnp.min(jnp.where(d == dmin[:, None], col, _K), axis=1)
    idx_ref[0, 0, :] = am
    acc_ref[0] += jnp.sum(dmin)
    onehot = (col == am[:, None]).astype(jnp.float32)
    counts_ref[...] += jnp.sum(onehot, axis=0, keepdims=True)

    @pl.when(i == _GRID - 1)
    def _fini():
        loss = (acc_ref[0] / (_N * _D)) * (1.0 + _COMMIT)
        loss_ref[...] = jnp.full((1, 1), loss, jnp.float32)
        p = counts_ref[0, :] * (1.0 / _N)
        ent = jnp.sum(p * jnp.log(p + 1e-10))
        perp_ref[...] = jnp.full((1, 1), jnp.exp(-ent), jnp.float32)


def _vq_argmin_call(flat_x, x2, emb, e2):
    return pl.pallas_call(
        _argmin_body,
        grid=(_GRID,),
        in_specs=[
            pl.BlockSpec((_TB, _D), lambda i: (i, 0)),
            pl.BlockSpec((_TB, 1), lambda i: (i, 0)),
            pl.BlockSpec((_K, _D), lambda i: (0, 0)),
            pl.BlockSpec((1, _K), lambda i: (0, 0)),
        ],
        out_specs=[
            pl.BlockSpec((1, 1, _TB), lambda i: (i, 0, 0)),
            pl.BlockSpec((1, 1), lambda i: (0, 0)),
            pl.BlockSpec((1, 1), lambda i: (0, 0)),
        ],
        out_shape=[
            jax.ShapeDtypeStruct((_GRID, 1, _TB), jnp.int32),
            jax.ShapeDtypeStruct((1, 1), jnp.float32),
            jax.ShapeDtypeStruct((1, 1), jnp.float32),
        ],
        scratch_shapes=[
            pltpu.VMEM((1, _K), jnp.float32),
            pltpu.SMEM((1,), jnp.float32),
        ],
        compiler_params=pltpu.CompilerParams(
            dimension_semantics=("arbitrary",)),
    )(flat_x, x2, emb, e2)


def _sc_gather_body(emb_hbm, idx_hbm, q_hbm, idx_v, rows_v, sem):
    wid = lax.axis_index("s") * _SC_CORES + lax.axis_index("c")
    base = wid * _BPW
    pltpu.sync_copy(idx_hbm.at[pl.ds(base, _BPW)], idx_v)
    copies = [
        pltpu.async_copy(
            emb_hbm.at[idx_v.at[pl.ds(j * _CHUNK, _CHUNK)]],
            rows_v.at[pl.ds(j * _CHUNK, _CHUNK)],
            sem)
        for j in range(_BPW // _CHUNK)
    ]
    for c in copies:
        c.wait()
    pltpu.sync_copy(rows_v, q_hbm.at[pl.ds(base, _BPW)])


@functools.lru_cache(maxsize=1)
def _sc_gather_call():
    return pl.kernel(
        _sc_gather_body,
        out_type=jax.ShapeDtypeStruct((_N, _D), jnp.float32),
        mesh=plsc.VectorSubcoreMesh(core_axis_name="c",
                                    subcore_axis_name="s"),
        scratch_types=[
            pltpu.VMEM((_BPW,), jnp.int32),
            pltpu.VMEM((_BPW, _D), jnp.float32),
            pltpu.SemaphoreType.DMA,
        ],
        compiler_params=pltpu.CompilerParams(use_tc_tiling_on_sc=False),
    )


def kernel(inputs, embedding_weight):
    x = jnp.transpose(inputs, (0, 2, 3, 1))
    input_shape = x.shape
    flat_x = x.reshape(-1, _D)
    x2 = jnp.sum(flat_x ** 2, axis=1, keepdims=True)
    e2 = jnp.sum(embedding_weight ** 2, axis=1).reshape(1, _K)

    idx3, loss, perp = _vq_argmin_call(flat_x, x2, embedding_weight, e2)
    idx = idx3.reshape(_N)

    quantized = _sc_gather_call()(embedding_weight, idx)

    quantized_out = jnp.transpose(quantized.reshape(input_shape), (0, 3, 1, 2))
    return (quantized_out, loss[0, 0], perp[0, 0],
            idx.reshape(input_shape[:-1]))
